# baseline (speedup 1.0000x reference)
import functools

import jax
import jax.numpy as jnp
from jax import lax
from jax.experimental import pallas as pl
from jax.experimental.pallas import tpu as pltpu
from jax.experimental.pallas import tpu_sc as plsc

_NC = 2
_NS = 16
_NW = _NC * _NS
_B, _T, _D = 64, 196, 768
_BPT = _B // _NW
_CW = 256
_CPB = _D // _CW
_NCHUNK = _BPT * _CPB

_mesh = plsc.VectorSubcoreMesh(core_axis_name="c", subcore_axis_name="s")


@functools.partial(
    pl.kernel,
    mesh=_mesh,
    out_type=jax.ShapeDtypeStruct((_B, _T, _D), jnp.float32),
    scratch_types=[
        pltpu.VMEM((_T, _CW), jnp.float32),
        pltpu.VMEM((_T, _CW), jnp.float32),
        pltpu.SemaphoreType.DMA,
        pltpu.SemaphoreType.DMA,
        pltpu.SemaphoreType.DMA,
        pltpu.SemaphoreType.DMA,
    ],
)
def _sc_copy(x_hbm, out_hbm, buf0, buf1, isem0, isem1, osem0, osem1):
    wid = lax.axis_index("s") * _NC + lax.axis_index("c")
    b0 = wid * _BPT
    bufs = (buf0, buf1)
    isems = (isem0, isem1)
    osems = (osem0, osem1)

    def src(k):
        return x_hbm.at[b0 + k // _CPB, :, pl.ds((k % _CPB) * _CW, _CW)]

    def dst(k):
        return out_hbm.at[b0 + k // _CPB, :, pl.ds((k % _CPB) * _CW, _CW)]

    pltpu.make_async_copy(src(0), bufs[0], isems[0]).start()
    for k in range(_NCHUNK):
        s = k % 2
        o = (k + 1) % 2
        pltpu.make_async_copy(src(k), bufs[s], isems[s]).wait()
        if k + 1 < _NCHUNK:
            if k >= 1:
                pltpu.make_async_copy(bufs[o], dst(k - 1), osems[o]).wait()
            pltpu.make_async_copy(src(k + 1), bufs[o], isems[o]).start()
        pltpu.make_async_copy(bufs[s], dst(k), osems[s]).start()
    pltpu.make_async_copy(bufs[(_NCHUNK - 2) % 2], dst(_NCHUNK - 2),
                          osems[(_NCHUNK - 2) % 2]).wait()
    pltpu.make_async_copy(bufs[(_NCHUNK - 1) % 2], dst(_NCHUNK - 1),
                          osems[(_NCHUNK - 1) % 2]).wait()


def kernel(x):
    return _sc_copy(x)
